# Initial kernel scaffold; baseline (speedup 1.0000x reference)
import jax
import jax.numpy as jnp
from jax.experimental import pallas as pl


def kernel(x, adj_indices, adj_values, W, b, W_out, b_out):
    raise NotImplementedError("write your pallas kernel here")


# speedup vs baseline: 6.7310x; 6.7310x over previous
import functools

import jax
import jax.numpy as jnp
from jax import lax
from jax.experimental import pallas as pl
from jax.experimental.pallas import tpu as pltpu
from jax.experimental.pallas import tpu_sc as plsc

N = 10000
E = 320000
K = 4
F = 128
H = 128
O = 64

NC = 2
NS = 16
CHUNK = 80
EPT = E // NS
NCH = EPT // CHUNK
RPT = 624
RTAIL = N - NS * RPT
HOPS = K // NC
VECS = CHUNK // 16
FV = H // 16


def _linear_tc(x, W, b):
  BN = 1000

  def body(x_ref, w_ref, b_ref, o_ref):
    o_ref[0] = (
        jnp.dot(x_ref[...], w_ref[0], preferred_element_type=jnp.float32)
        + b_ref[0]
    )

  return pl.pallas_call(
      body,
      grid=(K, N // BN),
      in_specs=[
          pl.BlockSpec((BN, F), lambda k, i: (i, 0)),
          pl.BlockSpec((1, F, H), lambda k, i: (k, 0, 0)),
          pl.BlockSpec((1, 1, H), lambda k, i: (k, 0, 0)),
      ],
      out_specs=pl.BlockSpec((1, BN, H), lambda k, i: (k, i, 0)),
      out_shape=jax.ShapeDtypeStruct((K, N, H), jnp.float32),
  )(x, W, b.reshape(K, 1, H))


def _out_tc(agg, Wr, b2):
  BN = 1000

  def body(a_ref, w_ref, b_ref, o_ref):
    acc = jnp.zeros((BN, O), jnp.float32)
    for k in range(K):
      a = a_ref[k]
      e = jnp.where(a > 0.0, a, jnp.exp(a) - 1.0)
      acc = acc + jnp.dot(e, w_ref[k], preferred_element_type=jnp.float32)
    o_ref[...] = acc + b_ref[...]

  return pl.pallas_call(
      body,
      grid=(N // BN,),
      in_specs=[
          pl.BlockSpec((K, BN, H), lambda i: (0, i, 0)),
          pl.BlockSpec((K, H, O), lambda i: (0, 0, 0)),
          pl.BlockSpec((1, O), lambda i: (0, 0)),
      ],
      out_specs=pl.BlockSpec((BN, O), lambda i: (i, 0)),
      out_shape=jax.ShapeDtypeStruct((N, O), jnp.float32),
  )(agg, Wr, b2)


def _spmm_sc(h_flat, rows, cols, vals, zeros):
  mesh = plsc.VectorSubcoreMesh(
      core_axis_name="c", subcore_axis_name="s",
      num_cores=NC, num_subcores=NS,
  )

  scratch = (
      [pltpu.VMEM((CHUNK,), jnp.int32) for _ in range(2)]
      + [pltpu.VMEM((CHUNK,), jnp.int32) for _ in range(2)]
      + [pltpu.VMEM((CHUNK,), jnp.float32) for _ in range(2)]
      + [pltpu.VMEM((CHUNK,), jnp.int32) for _ in range(2)]
      + [pltpu.VMEM((CHUNK,), jnp.int32) for _ in range(2)]
      + [pltpu.VMEM((CHUNK, H), jnp.float32) for _ in range(2)]
      + [pltpu.VMEM_SHARED((N, H), jnp.float32)]
      + [pltpu.SemaphoreType.DMA for _ in range(6)]
  )

  @functools.partial(
      pl.kernel,
      out_type=jax.ShapeDtypeStruct((K * N, H), jnp.float32),
      mesh=mesh,
      scratch_types=scratch,
  )
  def body(h_ref, rows_ref, cols_ref, vals_ref, z_ref, out_ref,
           col0, col1, row0, row1, val0, val1, gix0, gix1, six0, six1,
           gb0, gb1, agg, se0, se1, sg0, sg1, ss0, ss1):
    cid = lax.axis_index("c")
    sid = lax.axis_index("s")
    colb = (col0, col1)
    rowb = (row0, row1)
    valb = (val0, val1)
    gixb = (gix0, gix1)
    sixb = (six0, six1)
    gbb = (gb0, gb1)
    seme = (se0, se1)
    semg = (sg0, sg1)
    sems = (ss0, ss1)
    rs = sid * RPT

    for hi in range(HOPS):
      k = cid * HOPS + hi
      ebase = k * E + sid * EPT

      def fetch(c, bi):
        st = ebase + c * CHUNK
        pltpu.async_copy(rows_ref.at[pl.ds(st, CHUNK)], rowb[bi], seme[bi])
        pltpu.async_copy(cols_ref.at[pl.ds(st, CHUNK)], colb[bi], seme[bi])
        pltpu.async_copy(vals_ref.at[pl.ds(st, CHUNK)], valb[bi], seme[bi])

      def wait_fetch(c, bi):
        st = ebase + c * CHUNK
        pltpu.make_async_copy(
            rows_ref.at[pl.ds(st, CHUNK)], rowb[bi], seme[bi]).wait()
        pltpu.make_async_copy(
            cols_ref.at[pl.ds(st, CHUNK)], colb[bi], seme[bi]).wait()
        pltpu.make_async_copy(
            vals_ref.at[pl.ds(st, CHUNK)], valb[bi], seme[bi]).wait()

      def gidx_and_gather(bi):
        off = k * N
        for v in range(VECS):
          sl = pl.ds(v * 16, 16)
          gixb[bi][sl] = colb[bi][sl] + off
        pltpu.async_copy(h_ref.at[gixb[bi]], gbb[bi], semg[bi])

      def wait_gather(bi):
        pltpu.make_async_copy(h_ref.at[gixb[bi]], gbb[bi], semg[bi]).wait()

      def scale(bi):
        def gbody(g, carry):
          vv = valb[bi][pl.ds(g * 16, 16)]
          for j in range(16):
            vsp = jnp.full((16,), vv[j], jnp.float32)
            e = g * 16 + j
            for f in range(FV):
              sl = (e, pl.ds(f * 16, 16))
              gbb[bi][sl] = gbb[bi][sl] * vsp
          return carry
        lax.fori_loop(0, VECS, gbody, 0)
        for v in range(VECS):
          sl = pl.ds(v * 16, 16)
          sixb[bi][sl] = rowb[bi][sl]

      def scatter(bi):
        pltpu.async_copy(gbb[bi], agg.at[sixb[bi]], sems[bi], add=True)

      def wait_scatter(bi):
        pltpu.make_async_copy(gbb[bi], agg.at[sixb[bi]], sems[bi]).wait()

      def sub(c, bi):
        obi = 1 - bi
        wait_gather(bi)
        scale(bi)
        scatter(bi)

        @pl.when(c + 2 < NCH)
        def _():
          fetch(c + 2, bi)

        @pl.when(c + 1 < NCH)
        def _():
          wait_fetch(c + 1, obi)

          @pl.when(c >= 1)
          def _():
            wait_scatter(obi)

          gidx_and_gather(obi)

      fetch(jnp.int32(0), 0)
      fetch(jnp.int32(1), 1)
      pltpu.sync_copy(z_ref.at[pl.ds(rs, RPT)], agg.at[pl.ds(rs, RPT)])

      @pl.when(sid == 0)
      def _():
        pltpu.sync_copy(z_ref.at[pl.ds(NS * RPT, RTAIL)],
                        agg.at[pl.ds(NS * RPT, RTAIL)])

      plsc.subcore_barrier()
      wait_fetch(jnp.int32(0), 0)
      gidx_and_gather(0)

      def tbody(t, carry):
        sub(2 * t, 0)
        sub(2 * t + 1, 1)
        return carry

      lax.fori_loop(0, NCH // 2, tbody, 0)

      wait_scatter(0)
      wait_scatter(1)
      plsc.subcore_barrier()
      pltpu.sync_copy(agg.at[pl.ds(rs, RPT)], out_ref.at[pl.ds(k * N + rs, RPT)])

      @pl.when(sid == 0)
      def _():
        pltpu.sync_copy(agg.at[pl.ds(NS * RPT, RTAIL)],
                        out_ref.at[pl.ds(k * N + NS * RPT, RTAIL)])

      plsc.subcore_barrier()

  return body(h_flat, rows, cols, vals, zeros)


def kernel(x, adj_indices, adj_values, W, b, W_out, b_out):
  h_all = _linear_tc(x, W, b)
  h_flat = h_all.reshape(K * N, H)
  rows = adj_indices[:, 0, :].reshape(K * E)
  cols = adj_indices[:, 1, :].reshape(K * E)
  vals = adj_values.reshape(K * E)
  zeros = jnp.zeros((N, H), jnp.float32)
  agg = _spmm_sc(h_flat, rows, cols, vals, zeros).reshape(K, N, H)
  return _out_tc(agg, W_out.reshape(K, H, O), b_out.reshape(1, O))


# speedup vs baseline: 10.0427x; 1.4920x over previous
import functools

import jax
import jax.numpy as jnp
from jax import lax
from jax.experimental import pallas as pl
from jax.experimental.pallas import tpu as pltpu
from jax.experimental.pallas import tpu_sc as plsc

N = 10000
E = 320000
K = 4
F = 128
H = 128
O = 64

NC = 2
NS = 16
SUB = 80
NSUB = 2
CHUNK = SUB * NSUB
EPT = E // NS
NCH = EPT // CHUNK
RPT = 624
RTAIL = N - NS * RPT
HOPS = K // NC
VECS = CHUNK // 16
FV = H // 16


def _linear_tc(x, W, b):
  BN = 1000

  def body(x_ref, w_ref, b_ref, o_ref):
    o_ref[0] = (
        jnp.dot(x_ref[...], w_ref[0], preferred_element_type=jnp.float32)
        + b_ref[0]
    )

  return pl.pallas_call(
      body,
      grid=(K, N // BN),
      in_specs=[
          pl.BlockSpec((BN, F), lambda k, i: (i, 0)),
          pl.BlockSpec((1, F, H), lambda k, i: (k, 0, 0)),
          pl.BlockSpec((1, 1, H), lambda k, i: (k, 0, 0)),
      ],
      out_specs=pl.BlockSpec((1, BN, H), lambda k, i: (k, i, 0)),
      out_shape=jax.ShapeDtypeStruct((K, N, H), jnp.float32),
  )(x, W, b.reshape(K, 1, H))


def _out_tc(agg, Wr, b2):
  BN = 1000

  def body(a_ref, w_ref, b_ref, o_ref):
    acc = jnp.zeros((BN, O), jnp.float32)
    for k in range(K):
      a = a_ref[k]
      e = jnp.where(a > 0.0, a, jnp.exp(a) - 1.0)
      acc = acc + jnp.dot(e, w_ref[k], preferred_element_type=jnp.float32)
    o_ref[...] = acc + b_ref[...]

  return pl.pallas_call(
      body,
      grid=(N // BN,),
      in_specs=[
          pl.BlockSpec((K, BN, H), lambda i: (0, i, 0)),
          pl.BlockSpec((K, H, O), lambda i: (0, 0, 0)),
          pl.BlockSpec((1, O), lambda i: (0, 0)),
      ],
      out_specs=pl.BlockSpec((BN, O), lambda i: (i, 0)),
      out_shape=jax.ShapeDtypeStruct((N, O), jnp.float32),
  )(agg, Wr, b2)


def _spmm_sc(h_flat, rows, cols, vals, zeros):
  mesh = plsc.VectorSubcoreMesh(
      core_axis_name="c", subcore_axis_name="s",
      num_cores=NC, num_subcores=NS,
  )

  scratch = (
      [pltpu.VMEM((CHUNK,), jnp.int32) for _ in range(2)]
      + [pltpu.VMEM((CHUNK,), jnp.int32) for _ in range(2)]
      + [pltpu.VMEM((CHUNK,), jnp.float32) for _ in range(2)]
      + [pltpu.VMEM((CHUNK,), jnp.int32) for _ in range(2)]
      + [pltpu.VMEM((NSUB, SUB), jnp.int32) for _ in range(2)]
      + [pltpu.VMEM((CHUNK, H), jnp.float32) for _ in range(2)]
      + [pltpu.VMEM_SHARED((N, H), jnp.float32)]
      + [pltpu.SemaphoreType.DMA for _ in range(6)]
  )

  @functools.partial(
      pl.kernel,
      out_type=jax.ShapeDtypeStruct((K * N, H), jnp.float32),
      mesh=mesh,
      scratch_types=scratch,
  )
  def body(h_ref, rows_ref, cols_ref, vals_ref, z_ref, out_ref,
           col0, col1, row0, row1, val0, val1, gix0, gix1, six0, six1,
           gb0, gb1, agg, se0, se1, sg0, sg1, ss0, ss1):
    cid = lax.axis_index("c")
    sid = lax.axis_index("s")
    colb = (col0, col1)
    rowb = (row0, row1)
    valb = (val0, val1)
    gixb = (gix0, gix1)
    sixb = (six0, six1)
    gbb = (gb0, gb1)
    seme = (se0, se1)
    semg = (sg0, sg1)
    sems = (ss0, ss1)
    rs = sid * RPT

    for hi in range(HOPS):
      k = cid * HOPS + hi
      ebase = k * E + sid * EPT

      def fetch(c, bi):
        st = ebase + c * CHUNK
        pltpu.async_copy(rows_ref.at[pl.ds(st, CHUNK)], rowb[bi], seme[bi])
        pltpu.async_copy(cols_ref.at[pl.ds(st, CHUNK)], colb[bi], seme[bi])
        pltpu.async_copy(vals_ref.at[pl.ds(st, CHUNK)], valb[bi], seme[bi])

      def wait_fetch(c, bi):
        st = ebase + c * CHUNK
        pltpu.make_async_copy(
            rows_ref.at[pl.ds(st, CHUNK)], rowb[bi], seme[bi]).wait()
        pltpu.make_async_copy(
            cols_ref.at[pl.ds(st, CHUNK)], colb[bi], seme[bi]).wait()
        pltpu.make_async_copy(
            vals_ref.at[pl.ds(st, CHUNK)], valb[bi], seme[bi]).wait()

      def gidx_and_gather(bi):
        off = k * N

        @plsc.parallel_loop(0, VECS)
        def _(v):
          sl = pl.ds(v * 16, 16)
          gixb[bi][sl] = colb[bi][sl] + off

        for s in range(NSUB):
          sl = pl.ds(s * SUB, SUB)
          pltpu.async_copy(h_ref.at[gixb[bi].at[sl]], gbb[bi].at[sl],
                           semg[bi])

      def wait_gather(bi):
        for s in range(NSUB):
          sl = pl.ds(s * SUB, SUB)
          pltpu.make_async_copy(h_ref.at[gixb[bi].at[sl]], gbb[bi].at[sl],
                                semg[bi]).wait()

      def scale(bi):
        @plsc.parallel_loop(0, VECS)
        def _(g):
          vv = valb[bi][pl.ds(g * 16, 16)]
          gps = SUB // 16
          sixb[bi][g // gps, pl.ds((g % gps) * 16, 16)] = (
              rowb[bi][pl.ds(g * 16, 16)])
          for j in range(16):
            vsp = jnp.full((16,), vv[j], jnp.float32)
            e = g * 16 + j
            for f in range(FV):
              sl = (e, pl.ds(f * 16, 16))
              gbb[bi][sl] = gbb[bi][sl] * vsp

      def scatter(bi):
        for s in range(NSUB):
          pltpu.async_copy(gbb[bi].at[pl.ds(s * SUB, SUB)],
                           agg.at[sixb[bi].at[s]], sems[bi], add=True)

      def wait_scatter(bi):
        for s in range(NSUB):
          pltpu.make_async_copy(gbb[bi].at[pl.ds(s * SUB, SUB)],
                                agg.at[sixb[bi].at[s]], sems[bi]).wait()

      def stage(c, bi):
        obi = 1 - bi
        wait_gather(bi)

        @pl.when(c + 1 < NCH)
        def _():
          wait_fetch(c + 1, obi)

          @pl.when(c >= 1)
          def _():
            wait_scatter(obi)

          gidx_and_gather(obi)

        scale(bi)
        scatter(bi)

        @pl.when(c + 2 < NCH)
        def _():
          fetch(c + 2, bi)

      fetch(jnp.int32(0), 0)
      fetch(jnp.int32(1), 1)
      pltpu.sync_copy(z_ref.at[pl.ds(rs, RPT)], agg.at[pl.ds(rs, RPT)])

      @pl.when(sid == 0)
      def _():
        pltpu.sync_copy(z_ref.at[pl.ds(NS * RPT, RTAIL)],
                        agg.at[pl.ds(NS * RPT, RTAIL)])

      plsc.subcore_barrier()
      wait_fetch(jnp.int32(0), 0)
      gidx_and_gather(0)

      def tbody(t, carry):
        stage(2 * t, 0)
        stage(2 * t + 1, 1)
        return carry

      lax.fori_loop(0, NCH // 2, tbody, 0)
      if NCH % 2:
        stage(jnp.int32(NCH - 1), 0)

      wait_scatter(0)
      wait_scatter(1)
      plsc.subcore_barrier()
      pltpu.sync_copy(agg.at[pl.ds(rs, RPT)], out_ref.at[pl.ds(k * N + rs, RPT)])

      @pl.when(sid == 0)
      def _():
        pltpu.sync_copy(agg.at[pl.ds(NS * RPT, RTAIL)],
                        out_ref.at[pl.ds(k * N + NS * RPT, RTAIL)])

      plsc.subcore_barrier()

  return body(h_flat, rows, cols, vals, zeros)


def kernel(x, adj_indices, adj_values, W, b, W_out, b_out):
  h_all = _linear_tc(x, W, b)
  h_flat = h_all.reshape(K * N, H)
  rows = adj_indices[:, 0, :].reshape(K * E)
  cols = adj_indices[:, 1, :].reshape(K * E)
  vals = adj_values.reshape(K * E)
  zeros = jnp.zeros((N, H), jnp.float32)
  agg = _spmm_sc(h_flat, rows, cols, vals, zeros).reshape(K, N, H)
  return _out_tc(agg, W_out.reshape(K, H, O), b_out.reshape(1, O))
